# baseline (speedup 1.0000x reference)
import functools

import jax
import jax.numpy as jnp
from jax import lax
from jax.experimental import pallas as pl
from jax.experimental.pallas import tpu as pltpu
from jax.experimental.pallas import tpu_sc as plsc

N_NODES = 10000
N_EDGES = 320000
D = 128
N_GRAPHS = 128
N_CLASSES = 32

CHUNK = 128
N_TILES = 32
SUBCORES = 16
DEPTH = 3
BURSTS_PER_TILE = 81
EDGES_PAD = BURSTS_PER_TILE * N_TILES * CHUNK
ACC_ROWS = 10016
PAD_DST = 10008
ROWS_PER_TILE = 624
ROWS_TAIL = N_NODES - ROWS_PER_TILE * SUBCORES

_sc_mesh = plsc.VectorSubcoreMesh(core_axis_name="c", subcore_axis_name="s")


@functools.partial(
    pl.kernel,
    mesh=_sc_mesh,
    out_type=jax.ShapeDtypeStruct((2, N_NODES, D), jnp.float32),
    scratch_types=(
        [pltpu.VMEM((CHUNK,), jnp.int32) for _ in range(DEPTH)]
        + [pltpu.VMEM((CHUNK,), jnp.int32) for _ in range(DEPTH)]
        + [pltpu.VMEM((CHUNK, D), jnp.float32) for _ in range(DEPTH)]
        + [pltpu.VMEM_SHARED((ACC_ROWS, D), jnp.float32)]
        + [pltpu.SemaphoreType.DMA] * (3 * DEPTH)
    ),
)
def _edge_scatter_add(x_hbm, src_hbm, dst_hbm, out_hbm,
                      is0, is1, is2, id0, id1, id2, r0, r1, r2, acc_sh,
                      ss0, ss1, ss2, sd0, sd1, sd2, sg0, sg1, sg2):
    c = lax.axis_index("c")
    s = lax.axis_index("s")
    wid = s * 2 + c
    isrc = (is0, is1, is2)
    idst = (id0, id1, id2)
    rows = (r0, r1, r2)
    ssem = (ss0, ss1, ss2)
    dsem = (sd0, sd1, sd2)
    gsem = (sg0, sg1, sg2)
    e_base = wid * (BURSTS_PER_TILE * CHUNK)

    def _start_idx(j, b):
        e0 = e_base + j * CHUNK
        pltpu.async_copy(src_hbm.at[pl.ds(e0, CHUNK)], isrc[b], ssem[b])
        pltpu.async_copy(dst_hbm.at[pl.ds(e0, CHUNK)], idst[b], dsem[b])

    def _wait_idx(b):
        pltpu.make_async_copy(src_hbm.at[pl.ds(0, CHUNK)], isrc[b],
                              ssem[b]).wait()
        pltpu.make_async_copy(dst_hbm.at[pl.ds(0, CHUNK)], idst[b],
                              dsem[b]).wait()

    def _start_gather(b):
        pltpu.async_copy(x_hbm.at[isrc[b]], rows[b], gsem[b])

    def _wait_gather(b):
        pltpu.make_async_copy(x_hbm.at[pl.ds(0, CHUNK)], rows[b],
                              gsem[b]).wait()

    for b in range(DEPTH):
        _start_idx(b, b)

    def _zero_row(r, _):
        def _zero_lane(k, _):
            r0[r, pl.ds(k * 16, 16)] = jnp.zeros((16,), jnp.float32)
            return 0
        return lax.fori_loop(0, D // 16, _zero_lane, 0)
    lax.fori_loop(0, CHUNK, _zero_row, 0)

    base = s * ROWS_PER_TILE
    for j in range(ROWS_PER_TILE // CHUNK):
        pltpu.sync_copy(r0, acc_sh.at[pl.ds(base + j * CHUNK, CHUNK)])
    rem = ROWS_PER_TILE % CHUNK
    if rem:
        pltpu.sync_copy(
            r0.at[pl.ds(0, rem)],
            acc_sh.at[pl.ds(base + (ROWS_PER_TILE // CHUNK) * CHUNK, rem)])

    @pl.when(s == 0)
    def _():
        pltpu.sync_copy(
            r0.at[pl.ds(0, ACC_ROWS - ROWS_PER_TILE * SUBCORES)],
            acc_sh.at[pl.ds(ROWS_PER_TILE * SUBCORES,
                            ACC_ROWS - ROWS_PER_TILE * SUBCORES)])

    _wait_idx(0)
    _start_gather(0)
    plsc.subcore_barrier()

    def _round(i, _):
        for b in range(DEPTH):
            j = i * DEPTH + b
            b1 = (b + 1) % DEPTH
            _wait_gather(b)

            @pl.when(j + 1 < BURSTS_PER_TILE)
            def _():
                _wait_idx(b1)
                _start_gather(b1)

            pltpu.sync_copy(rows[b], acc_sh.at[idst[b]], add=True)

            @pl.when(j + DEPTH < BURSTS_PER_TILE)
            def _():
                _start_idx(j + DEPTH, b)
        return 0
    lax.fori_loop(0, BURSTS_PER_TILE // DEPTH, _round, 0)

    plsc.subcore_barrier()
    pltpu.sync_copy(acc_sh.at[pl.ds(base, ROWS_PER_TILE)],
                    out_hbm.at[c, pl.ds(base, ROWS_PER_TILE)])

    @pl.when(s == 0)
    def _():
        pltpu.sync_copy(
            acc_sh.at[pl.ds(ROWS_PER_TILE * SUBCORES, ROWS_TAIL)],
            out_hbm.at[c, pl.ds(ROWS_PER_TILE * SUBCORES, ROWS_TAIL)])


ROWS_B = 1000


def _mlp_body(x_ref, a0_ref, a1_ref, wa_ref, ba_ref, wb_ref, bb_ref, o_ref):
    h = x_ref[...] + a0_ref[...] + a1_ref[...]
    h = jnp.dot(h, wa_ref[...], preferred_element_type=jnp.float32) + ba_ref[...]
    h = jnp.maximum(h, 0.0)
    h = jnp.dot(h, wb_ref[...], preferred_element_type=jnp.float32) + bb_ref[...]
    o_ref[...] = jnp.maximum(h, 0.0)


def _mlp(x, a0, a1, wa, ba, wb, bb):
    row_spec = pl.BlockSpec((ROWS_B, D), lambda i: (i, 0))
    w_spec = pl.BlockSpec((D, D), lambda i: (0, 0))
    b_spec = pl.BlockSpec((1, D), lambda i: (0, 0))
    return pl.pallas_call(
        _mlp_body,
        grid=(N_NODES // ROWS_B,),
        in_specs=[row_spec, row_spec, row_spec, w_spec, b_spec, w_spec, b_spec],
        out_specs=row_spec,
        out_shape=jax.ShapeDtypeStruct((N_NODES, D), jnp.float32),
    )(x, a0, a1, wa, ba.reshape(1, D), wb, bb.reshape(1, D))


def _mlp_pool_body(x_ref, a0_ref, a1_ref, wa_ref, ba_ref, wb_ref, bb_ref,
                   batch_ref, o_ref):
    h = x_ref[...] + a0_ref[...] + a1_ref[...]
    h = jnp.dot(h, wa_ref[...], preferred_element_type=jnp.float32) + ba_ref[...]
    h = jnp.maximum(h, 0.0)
    h = jnp.dot(h, wb_ref[...], preferred_element_type=jnp.float32) + bb_ref[...]
    h = jnp.maximum(h, 0.0)
    onehot = (batch_ref[...] == lax.broadcasted_iota(
        jnp.int32, (ROWS_B, N_GRAPHS), 1)).astype(jnp.float32)
    part = lax.dot_general(onehot, h, (((0,), (0,)), ((), ())),
                           preferred_element_type=jnp.float32)

    @pl.when(pl.program_id(0) == 0)
    def _():
        o_ref[...] = part

    @pl.when(pl.program_id(0) > 0)
    def _():
        o_ref[...] += part


def _mlp_pool(x, a0, a1, wa, ba, wb, bb, batch2):
    row_spec = pl.BlockSpec((ROWS_B, D), lambda i: (i, 0))
    w_spec = pl.BlockSpec((D, D), lambda i: (0, 0))
    b_spec = pl.BlockSpec((1, D), lambda i: (0, 0))
    return pl.pallas_call(
        _mlp_pool_body,
        grid=(N_NODES // ROWS_B,),
        in_specs=[row_spec, row_spec, row_spec, w_spec, b_spec, w_spec, b_spec,
                  pl.BlockSpec((ROWS_B, 1), lambda i: (i, 0))],
        out_specs=pl.BlockSpec((N_GRAPHS, N_GRAPHS), lambda i: (0, 0)),
        out_shape=jax.ShapeDtypeStruct((N_GRAPHS, N_GRAPHS), jnp.float32),
    )(x, a0, a1, wa, ba.reshape(1, D), wb, bb.reshape(1, D), batch2)


def _head_body(p_ref, w1_ref, b1_ref, w2_ref, b2_ref, o_ref):
    h = jnp.dot(p_ref[...], w1_ref[...], preferred_element_type=jnp.float32)
    h = jnp.maximum(h + b1_ref[...], 0.0)
    z = jnp.dot(h, w2_ref[...], preferred_element_type=jnp.float32) + b2_ref[...]
    m = jnp.max(z, axis=1, keepdims=True)
    e = jnp.exp(z - m)
    o_ref[...] = z - m - jnp.log(jnp.sum(e, axis=1, keepdims=True))


def _head(pooled, w1, b1, w2, b2):
    return pl.pallas_call(
        _head_body,
        out_shape=jax.ShapeDtypeStruct((N_GRAPHS, N_CLASSES), jnp.float32),
    )(pooled, w1, b1.reshape(1, D), w2, b2.reshape(1, N_CLASSES))


def kernel(x, edge_index, batch, W1a, b1a, W1b, b1b, W2a, b2a, W2b, b2b,
           Wl1, bl1, Wl2, bl2):
    n_pad = EDGES_PAD - N_EDGES
    src = jnp.concatenate(
        [edge_index[0].astype(jnp.int32), jnp.zeros((n_pad,), jnp.int32)])
    dst = jnp.concatenate(
        [edge_index[1].astype(jnp.int32), jnp.full((n_pad,), PAD_DST, jnp.int32)])
    batch2 = batch.astype(jnp.int32).reshape(N_NODES, 1)

    agg1 = _edge_scatter_add(x, src, dst)
    h1 = _mlp(x, agg1[0], agg1[1], W1a, b1a, W1b, b1b)
    agg2 = _edge_scatter_add(h1, src, dst)
    pooled = _mlp_pool(h1, agg2[0], agg2[1], W2a, b2a, W2b, b2b, batch2)
    return _head(pooled, Wl1, bl1, Wl2, bl2)


# speedup vs baseline: 1.1598x; 1.1598x over previous
import functools

import jax
import jax.numpy as jnp
from jax import lax
from jax.experimental import pallas as pl
from jax.experimental.pallas import tpu as pltpu
from jax.experimental.pallas import tpu_sc as plsc

N_NODES = 10000
N_EDGES = 320000
D = 128
N_GRAPHS = 128
N_CLASSES = 32

CHUNK = 128
N_TILES = 32
SUBCORES = 16
BURSTS_PER_TILE = 80
GROUP = 8
GROUPS = BURSTS_PER_TILE // GROUP
EDGE_ROWS = BURSTS_PER_TILE * N_TILES + 2 * GROUP
ACC_ROWS = 10016
PAD_DST = 10008
ROWS_PER_TILE = 624
ROWS_TAIL = N_NODES - ROWS_PER_TILE * SUBCORES

_sc_mesh = plsc.VectorSubcoreMesh(core_axis_name="c", subcore_axis_name="s")


@functools.partial(
    pl.kernel,
    mesh=_sc_mesh,
    out_type=jax.ShapeDtypeStruct((2, N_NODES, D), jnp.float32),
    scratch_types=(
        [pltpu.VMEM((GROUP, CHUNK), jnp.int32) for _ in range(2)]
        + [pltpu.VMEM((GROUP, CHUNK), jnp.int32) for _ in range(2)]
        + [pltpu.VMEM((CHUNK, D), jnp.float32) for _ in range(2)]
        + [pltpu.VMEM_SHARED((ACC_ROWS, D), jnp.float32)]
        + [pltpu.SemaphoreType.DMA] * 6
    ),
)
def _edge_scatter_add(x_hbm, src_hbm, dst_hbm, out_hbm,
                      isA, isB, idA, idB, r0, r1, acc_sh,
                      ssA, ssB, sdA, sdB, sg0, sg1):
    c = lax.axis_index("c")
    s = lax.axis_index("s")
    wid = s * 2 + c
    srcg = (isA, isB)
    dstg = (idA, idB)
    rows = (r0, r1)
    ssem = (ssA, ssB)
    dsem = (sdA, sdB)
    gsem = (sg0, sg1)
    row_base = wid * BURSTS_PER_TILE

    def _start_idxgrp(g, P):
        r0w = row_base + g * GROUP
        pltpu.async_copy(src_hbm.at[pl.ds(r0w, GROUP)], srcg[P], ssem[P])
        pltpu.async_copy(dst_hbm.at[pl.ds(r0w, GROUP)], dstg[P], dsem[P])

    def _wait_idxgrp(P):
        pltpu.make_async_copy(src_hbm.at[pl.ds(0, GROUP)], srcg[P],
                              ssem[P]).wait()
        pltpu.make_async_copy(dst_hbm.at[pl.ds(0, GROUP)], dstg[P],
                              dsem[P]).wait()

    def _start_gather(idx_ref, b):
        pltpu.async_copy(x_hbm.at[idx_ref], rows[b], gsem[b])

    def _wait_gather(b):
        pltpu.make_async_copy(x_hbm.at[pl.ds(0, CHUNK)], rows[b],
                              gsem[b]).wait()

    _start_idxgrp(0, 0)
    _start_idxgrp(1, 1)

    def _zero_row(r, _):
        def _zero_lane(k, _):
            r0[r, pl.ds(k * 16, 16)] = jnp.zeros((16,), jnp.float32)
            return 0
        return lax.fori_loop(0, D // 16, _zero_lane, 0)
    lax.fori_loop(0, CHUNK, _zero_row, 0)

    base = s * ROWS_PER_TILE
    for j in range(ROWS_PER_TILE // CHUNK):
        pltpu.sync_copy(r0, acc_sh.at[pl.ds(base + j * CHUNK, CHUNK)])
    rem = ROWS_PER_TILE % CHUNK
    if rem:
        pltpu.sync_copy(
            r0.at[pl.ds(0, rem)],
            acc_sh.at[pl.ds(base + (ROWS_PER_TILE // CHUNK) * CHUNK, rem)])

    @pl.when(s == 0)
    def _():
        pltpu.sync_copy(
            r0.at[pl.ds(0, ACC_ROWS - ROWS_PER_TILE * SUBCORES)],
            acc_sh.at[pl.ds(ROWS_PER_TILE * SUBCORES,
                            ACC_ROWS - ROWS_PER_TILE * SUBCORES)])

    _wait_idxgrp(0)
    _start_gather(isA.at[0], 0)
    plsc.subcore_barrier()

    def _grp_pair(i, _):
        for P in range(2):
            for k in range(GROUP):
                b = k % 2
                _wait_gather(b)
                if k < GROUP - 1:
                    _start_gather(srcg[P].at[k + 1], 1 - b)
                else:
                    _wait_idxgrp(1 - P)
                    _start_gather(srcg[1 - P].at[0], 1 - b)
                pltpu.sync_copy(rows[b], acc_sh.at[dstg[P].at[k]], add=True)
            g = i * 2 + P
            _start_idxgrp(g + 2, P)
        return 0
    lax.fori_loop(0, GROUPS // 2, _grp_pair, 0)

    _wait_gather(0)
    _wait_idxgrp(1)

    plsc.subcore_barrier()
    pltpu.sync_copy(acc_sh.at[pl.ds(base, ROWS_PER_TILE)],
                    out_hbm.at[c, pl.ds(base, ROWS_PER_TILE)])

    @pl.when(s == 0)
    def _():
        pltpu.sync_copy(
            acc_sh.at[pl.ds(ROWS_PER_TILE * SUBCORES, ROWS_TAIL)],
            out_hbm.at[c, pl.ds(ROWS_PER_TILE * SUBCORES, ROWS_TAIL)])


ROWS_B = 1000


def _mlp_body(x_ref, a0_ref, a1_ref, wa_ref, ba_ref, wb_ref, bb_ref, o_ref):
    h = x_ref[...] + a0_ref[...] + a1_ref[...]
    h = jnp.dot(h, wa_ref[...], preferred_element_type=jnp.float32) + ba_ref[...]
    h = jnp.maximum(h, 0.0)
    h = jnp.dot(h, wb_ref[...], preferred_element_type=jnp.float32) + bb_ref[...]
    o_ref[...] = jnp.maximum(h, 0.0)


def _mlp(x, a0, a1, wa, ba, wb, bb):
    row_spec = pl.BlockSpec((ROWS_B, D), lambda i: (i, 0))
    w_spec = pl.BlockSpec((D, D), lambda i: (0, 0))
    b_spec = pl.BlockSpec((1, D), lambda i: (0, 0))
    return pl.pallas_call(
        _mlp_body,
        grid=(N_NODES // ROWS_B,),
        in_specs=[row_spec, row_spec, row_spec, w_spec, b_spec, w_spec, b_spec],
        out_specs=row_spec,
        out_shape=jax.ShapeDtypeStruct((N_NODES, D), jnp.float32),
    )(x, a0, a1, wa, ba.reshape(1, D), wb, bb.reshape(1, D))


def _mlp_pool_body(x_ref, a0_ref, a1_ref, wa_ref, ba_ref, wb_ref, bb_ref,
                   batch_ref, o_ref):
    h = x_ref[...] + a0_ref[...] + a1_ref[...]
    h = jnp.dot(h, wa_ref[...], preferred_element_type=jnp.float32) + ba_ref[...]
    h = jnp.maximum(h, 0.0)
    h = jnp.dot(h, wb_ref[...], preferred_element_type=jnp.float32) + bb_ref[...]
    h = jnp.maximum(h, 0.0)
    onehot = (batch_ref[...] == lax.broadcasted_iota(
        jnp.int32, (ROWS_B, N_GRAPHS), 1)).astype(jnp.float32)
    part = lax.dot_general(onehot, h, (((0,), (0,)), ((), ())),
                           preferred_element_type=jnp.float32)

    @pl.when(pl.program_id(0) == 0)
    def _():
        o_ref[...] = part

    @pl.when(pl.program_id(0) > 0)
    def _():
        o_ref[...] += part


def _mlp_pool(x, a0, a1, wa, ba, wb, bb, batch2):
    row_spec = pl.BlockSpec((ROWS_B, D), lambda i: (i, 0))
    w_spec = pl.BlockSpec((D, D), lambda i: (0, 0))
    b_spec = pl.BlockSpec((1, D), lambda i: (0, 0))
    return pl.pallas_call(
        _mlp_pool_body,
        grid=(N_NODES // ROWS_B,),
        in_specs=[row_spec, row_spec, row_spec, w_spec, b_spec, w_spec, b_spec,
                  pl.BlockSpec((ROWS_B, 1), lambda i: (i, 0))],
        out_specs=pl.BlockSpec((N_GRAPHS, N_GRAPHS), lambda i: (0, 0)),
        out_shape=jax.ShapeDtypeStruct((N_GRAPHS, N_GRAPHS), jnp.float32),
    )(x, a0, a1, wa, ba.reshape(1, D), wb, bb.reshape(1, D), batch2)


def _head_body(p_ref, w1_ref, b1_ref, w2_ref, b2_ref, o_ref):
    h = jnp.dot(p_ref[...], w1_ref[...], preferred_element_type=jnp.float32)
    h = jnp.maximum(h + b1_ref[...], 0.0)
    z = jnp.dot(h, w2_ref[...], preferred_element_type=jnp.float32) + b2_ref[...]
    m = jnp.max(z, axis=1, keepdims=True)
    e = jnp.exp(z - m)
    o_ref[...] = z - m - jnp.log(jnp.sum(e, axis=1, keepdims=True))


def _head(pooled, w1, b1, w2, b2):
    return pl.pallas_call(
        _head_body,
        out_shape=jax.ShapeDtypeStruct((N_GRAPHS, N_CLASSES), jnp.float32),
    )(pooled, w1, b1.reshape(1, D), w2, b2.reshape(1, N_CLASSES))


def kernel(x, edge_index, batch, W1a, b1a, W1b, b1b, W2a, b2a, W2b, b2b,
           Wl1, bl1, Wl2, bl2):
    n_pad = EDGE_ROWS * CHUNK - N_EDGES
    src = jnp.concatenate(
        [edge_index[0].astype(jnp.int32), jnp.zeros((n_pad,), jnp.int32)]
    ).reshape(EDGE_ROWS, CHUNK)
    dst = jnp.concatenate(
        [edge_index[1].astype(jnp.int32), jnp.full((n_pad,), PAD_DST, jnp.int32)]
    ).reshape(EDGE_ROWS, CHUNK)
    batch2 = batch.astype(jnp.int32).reshape(N_NODES, 1)

    agg1 = _edge_scatter_add(x, src, dst)
    h1 = _mlp(x, agg1[0], agg1[1], W1a, b1a, W1b, b1b)
    agg2 = _edge_scatter_add(h1, src, dst)
    pooled = _mlp_pool(h1, agg2[0], agg2[1], W2a, b2a, W2b, b2b, batch2)
    return _head(pooled, Wl1, bl1, Wl2, bl2)
